# Initial kernel scaffold; baseline (speedup 1.0000x reference)
import jax
import jax.numpy as jnp
from jax.experimental import pallas as pl


def kernel(x, edge_index, W1, b1, W2, b2):
    raise NotImplementedError("write your pallas kernel here")


# speedup vs baseline: 32.1324x; 32.1324x over previous
import functools

import jax
import jax.numpy as jnp
from jax import lax
from jax.experimental import pallas as pl
from jax.experimental.pallas import tpu as pltpu
from jax.experimental.pallas import tpu_sc as plsc

NC = 2
NS = 16
NW = NC * NS
B = 128


@functools.cache
def _build(N, E, D, H, C):
    assert H == 16, "row width must match the 64B DMA granule"
    Hp = 16
    NP = N + 16
    S = -(-E // (NW * B))
    S = S + (S % 2)
    EP = S * NW * B

    mesh = plsc.VectorSubcoreMesh(core_axis_name="c", subcore_axis_name="s")
    sc_params = pltpu.CompilerParams(use_tc_tiling_on_sc=False)

    @functools.partial(
        pl.kernel,
        out_type=jax.ShapeDtypeStruct((NC, NP), jnp.float32),
        mesh=mesh,
        compiler_params=sc_params,
        scratch_types=[
            pltpu.VMEM((S, B), jnp.int32),
            pltpu.VMEM((B,), jnp.float32),
            pltpu.VMEM_SHARED((NP,), jnp.float32),
        ],
    )
    def deg_kernel(dst_hbm, zn_hbm, out_hbm, dst_v, ones_v, dacc_sh):
        cid = lax.axis_index("c")
        sid = lax.axis_index("s")
        wid = sid * NC + cid
        pltpu.sync_copy(dst_hbm.at[wid], dst_v)
        for i in range(B // 16):
            ones_v[pl.ds(i * 16, 16)] = jnp.ones((16,), jnp.float32)

        @pl.when(sid == 0)
        def _init():
            pltpu.sync_copy(zn_hbm, dacc_sh)

        plsc.subcore_barrier()

        def body(j, carry):
            pltpu.sync_copy(ones_v, dacc_sh.at[dst_v.at[j]], add=True)
            return carry

        lax.fori_loop(0, S, body, 0)
        plsc.subcore_barrier()

        @pl.when(sid == 0)
        def _out():
            pltpu.sync_copy(dacc_sh, out_hbm.at[cid])

    @functools.partial(
        pl.kernel,
        out_type=jax.ShapeDtypeStruct((NC, NP, Hp), jnp.float32),
        mesh=mesh,
        compiler_params=sc_params,
        scratch_types=[
            pltpu.VMEM((S, B), jnp.int32),
            pltpu.VMEM((S, B), jnp.int32),
            pltpu.VMEM((B, Hp), jnp.float32),
            pltpu.VMEM_SHARED((NP, Hp), jnp.float32),
            pltpu.SemaphoreType.DMA,
        ],
    )
    def edge_kernel(src_hbm, dst_hbm, y_hbm, z_hbm, out_hbm,
                    src_v, dst_v, rows_v, acc_sh, gsem):
        cid = lax.axis_index("c")
        sid = lax.axis_index("s")
        wid = sid * NC + cid
        pltpu.sync_copy(src_hbm.at[wid], src_v)
        pltpu.sync_copy(dst_hbm.at[wid], dst_v)

        @pl.when(sid == 0)
        def _init():
            pltpu.sync_copy(z_hbm, acc_sh)

        plsc.subcore_barrier()

        def body(j, carry):
            pltpu.async_copy(y_hbm.at[src_v.at[j]], rows_v, gsem).wait()
            pltpu.sync_copy(rows_v, acc_sh.at[dst_v.at[j]], add=True)
            return carry

        lax.fori_loop(0, S, body, 0)
        plsc.subcore_barrier()

        @pl.when(sid == 0)
        def _out():
            pltpu.sync_copy(acc_sh, out_hbm.at[cid])

    def mm_body(x_ref, w_ref, o_ref):
        o_ref[...] = jnp.dot(x_ref[...], w_ref[...],
                             preferred_element_type=jnp.float32)

    mm1 = pl.pallas_call(
        mm_body, out_shape=jax.ShapeDtypeStruct((N, Hp), jnp.float32))

    def scale_body(xw_ref, d0_ref, d1_ref, y_ref, dinv_ref):
        dinv = lax.rsqrt(d0_ref[...] + d1_ref[...] + 1.0)
        y_ref[...] = xw_ref[...] * dinv
        dinv_ref[...] = dinv

    scale = pl.pallas_call(
        scale_body,
        out_shape=(jax.ShapeDtypeStruct((N, Hp), jnp.float32),
                   jax.ShapeDtypeStruct((N, 1), jnp.float32)))

    def mid_body(a0_ref, a1_ref, y1_ref, dinv_ref, b1_ref, w2_ref, y2_ref):
        dinv = dinv_ref[...]
        h = dinv * (a0_ref[...] + a1_ref[...] + y1_ref[...]) + b1_ref[...]
        h = jnp.maximum(h, 0.0)
        y2_ref[...] = jnp.dot(h, w2_ref[...],
                              preferred_element_type=jnp.float32) * dinv

    mid = pl.pallas_call(
        mid_body, out_shape=jax.ShapeDtypeStruct((N, Hp), jnp.float32))

    def fin_body(a0_ref, a1_ref, y2_ref, dinv_ref, b2_ref, o_ref):
        o = dinv_ref[...] * (a0_ref[...] + a1_ref[...] + y2_ref[...]) + b2_ref[...]
        col = lax.broadcasted_iota(jnp.int32, o.shape, 1)
        om = jnp.where(col < C, o, -jnp.inf)
        m = jnp.max(om, axis=1, keepdims=True)
        s = jnp.sum(jnp.where(col < C, jnp.exp(o - m), 0.0), axis=1,
                    keepdims=True)
        o_ref[...] = o - (m + jnp.log(s))

    fin = pl.pallas_call(
        fin_body, out_shape=jax.ShapeDtypeStruct((N, Hp), jnp.float32))

    return deg_kernel, edge_kernel, mm1, scale, mid, fin, NP, S, EP


def kernel(x, edge_index, W1, b1, W2, b2):
    N, D = x.shape
    E = edge_index.shape[1]
    H = W1.shape[1]
    C = W2.shape[1]
    Hp = 16
    (deg_kernel, edge_kernel, mm1, scale, mid, fin,
     NP, S, EP) = _build(N, E, D, H, C)

    src = edge_index[0].astype(jnp.int32)
    dst = edge_index[1].astype(jnp.int32)
    pad = EP - E
    lanes = jnp.arange(pad, dtype=jnp.int32) % 16
    srcp = jnp.concatenate([src, lanes]).reshape(NW, S, B)
    dstp = jnp.concatenate([dst, N + lanes]).reshape(NW, S, B)

    zn = jnp.zeros((NP,), jnp.float32)
    zr = jnp.zeros((NP, Hp), jnp.float32)
    W2p = jnp.concatenate(
        [W2, jnp.zeros((H, Hp - C), jnp.float32)], axis=1)
    b1r = b1.reshape(1, H)
    b2p = jnp.concatenate([b2, jnp.zeros((Hp - C,), jnp.float32)]).reshape(1, Hp)

    dpart = deg_kernel(dstp, zn)
    xw = mm1(x, W1)
    y1, dinv = scale(xw, dpart[0, :N, None], dpart[1, :N, None])
    a = edge_kernel(srcp, dstp, y1, zr)
    y2 = mid(a[0, :N], a[1, :N], y1, dinv, b1r, W2p)
    a2 = edge_kernel(srcp, dstp, y2, zr)
    out = fin(a2[0, :N], a2[1, :N], y2, dinv, b2p)
    return out[:, :C]


# speedup vs baseline: 44.8625x; 1.3962x over previous
import functools

import jax
import jax.numpy as jnp
from jax import lax
from jax.experimental import pallas as pl
from jax.experimental.pallas import tpu as pltpu
from jax.experimental.pallas import tpu_sc as plsc

NC = 2
NS = 16
NW = NC * NS
B = 128
NBUF = 4
GAHEAD = 2


@functools.cache
def _build(N, E, D, H, C):
    assert H == 16, "row width must match the 64B DMA granule"
    Hp = 16
    NP = -(-(N + 16) // 128) * 128
    CHUNK = NP // NS
    S = -(-E // (NW * B))
    S = max(S + (S % 2), 2 * NBUF)
    EP = S * NW * B

    mesh = plsc.VectorSubcoreMesh(core_axis_name="c", subcore_axis_name="s")
    sc_params = pltpu.CompilerParams(use_tc_tiling_on_sc=False)

    @functools.partial(
        pl.kernel,
        out_type=jax.ShapeDtypeStruct((NC, NP), jnp.float32),
        mesh=mesh,
        compiler_params=sc_params,
        scratch_types=[
            pltpu.VMEM((S, B), jnp.int32),
            pltpu.VMEM((B,), jnp.float32),
            pltpu.VMEM_SHARED((NP,), jnp.float32),
            pltpu.SemaphoreType.DMA((NBUF,)),
        ],
    )
    def deg_kernel(dst_hbm, zn_hbm, out_hbm, dst_v, ones_v, dacc_sh, ssem):
        cid = lax.axis_index("c")
        sid = lax.axis_index("s")
        wid = sid * NC + cid
        pltpu.sync_copy(dst_hbm.at[wid], dst_v)
        for i in range(B // 16):
            ones_v[pl.ds(i * 16, 16)] = jnp.ones((16,), jnp.float32)
        pltpu.sync_copy(zn_hbm.at[pl.ds(sid * CHUNK, CHUNK)],
                        dacc_sh.at[pl.ds(sid * CHUNK, CHUNK)])
        plsc.subcore_barrier()

        def body(j, carry):
            b = lax.rem(j, NBUF)

            @pl.when(j >= NBUF)
            def _wait_prev():
                pltpu.make_async_copy(
                    ones_v, dacc_sh.at[dst_v.at[j]], ssem.at[b]).wait()

            pltpu.async_copy(ones_v, dacc_sh.at[dst_v.at[j]], ssem.at[b],
                             add=True)
            return carry

        lax.fori_loop(0, S, body, 0)
        for b in range(NBUF):
            pltpu.make_async_copy(
                ones_v, dacc_sh.at[dst_v.at[0]], ssem.at[b]).wait()
        plsc.subcore_barrier()
        pltpu.sync_copy(dacc_sh.at[pl.ds(sid * CHUNK, CHUNK)],
                        out_hbm.at[cid, pl.ds(sid * CHUNK, CHUNK)])

    @functools.partial(
        pl.kernel,
        out_type=jax.ShapeDtypeStruct((NC, NP, Hp), jnp.float32),
        mesh=mesh,
        compiler_params=sc_params,
        scratch_types=[
            pltpu.VMEM((S, B), jnp.int32),
            pltpu.VMEM((S, B), jnp.int32),
            pltpu.VMEM((NBUF, B, Hp), jnp.float32),
            pltpu.VMEM_SHARED((NP, Hp), jnp.float32),
            pltpu.SemaphoreType.DMA((NBUF,)),
            pltpu.SemaphoreType.DMA((NBUF,)),
        ],
    )
    def edge_kernel(src_hbm, dst_hbm, y_hbm, z_hbm, out_hbm,
                    src_v, dst_v, rows_v, acc_sh, gsem, ssem):
        cid = lax.axis_index("c")
        sid = lax.axis_index("s")
        wid = sid * NC + cid
        pltpu.sync_copy(src_hbm.at[wid], src_v)
        pltpu.sync_copy(dst_hbm.at[wid], dst_v)
        pltpu.sync_copy(z_hbm.at[pl.ds(sid * CHUNK, CHUNK)],
                        acc_sh.at[pl.ds(sid * CHUNK, CHUNK)])
        plsc.subcore_barrier()

        def body(j, carry):
            @pl.when(j < S)
            def _gather():
                b = lax.rem(j, NBUF)

                @pl.when(j >= NBUF)
                def _wait_scatter():
                    pltpu.make_async_copy(
                        rows_v.at[b], acc_sh.at[dst_v.at[0]], ssem.at[b]).wait()

                pltpu.async_copy(
                    y_hbm.at[src_v.at[j]], rows_v.at[b], gsem.at[b])

            jj = j - GAHEAD

            @pl.when(jj >= 0)
            def _scatter():
                bb = lax.rem(jj, NBUF)
                pltpu.make_async_copy(
                    y_hbm.at[src_v.at[jj]], rows_v.at[bb], gsem.at[bb]).wait()
                pltpu.async_copy(
                    rows_v.at[bb], acc_sh.at[dst_v.at[jj]], ssem.at[bb],
                    add=True)

            return carry

        lax.fori_loop(0, S + GAHEAD, body, 0)
        for b in range(NBUF):
            pltpu.make_async_copy(
                rows_v.at[b], acc_sh.at[dst_v.at[0]], ssem.at[b]).wait()
        plsc.subcore_barrier()
        pltpu.sync_copy(acc_sh.at[pl.ds(sid * CHUNK, CHUNK)],
                        out_hbm.at[cid, pl.ds(sid * CHUNK, CHUNK)])

    def mm_body(x_ref, w_ref, o_ref):
        o_ref[...] = jnp.dot(x_ref[...], w_ref[...],
                             preferred_element_type=jnp.float32)

    mm1 = pl.pallas_call(
        mm_body, out_shape=jax.ShapeDtypeStruct((N, Hp), jnp.float32))

    def scale_body(xw_ref, d0_ref, d1_ref, y_ref, dinv_ref):
        dinv = lax.rsqrt(d0_ref[...] + d1_ref[...] + 1.0)
        y_ref[...] = xw_ref[...] * dinv
        dinv_ref[...] = dinv

    scale = pl.pallas_call(
        scale_body,
        out_shape=(jax.ShapeDtypeStruct((N, Hp), jnp.float32),
                   jax.ShapeDtypeStruct((N, 1), jnp.float32)))

    def mid_body(a0_ref, a1_ref, y1_ref, dinv_ref, b1_ref, w2_ref, y2_ref):
        dinv = dinv_ref[...]
        h = dinv * (a0_ref[...] + a1_ref[...] + y1_ref[...]) + b1_ref[...]
        h = jnp.maximum(h, 0.0)
        y2_ref[...] = jnp.dot(h, w2_ref[...],
                              preferred_element_type=jnp.float32) * dinv

    mid = pl.pallas_call(
        mid_body, out_shape=jax.ShapeDtypeStruct((N, Hp), jnp.float32))

    def fin_body(a0_ref, a1_ref, y2_ref, dinv_ref, b2_ref, o_ref):
        o = dinv_ref[...] * (a0_ref[...] + a1_ref[...] + y2_ref[...]) + b2_ref[...]
        col = lax.broadcasted_iota(jnp.int32, o.shape, 1)
        om = jnp.where(col < C, o, -jnp.inf)
        m = jnp.max(om, axis=1, keepdims=True)
        s = jnp.sum(jnp.where(col < C, jnp.exp(o - m), 0.0), axis=1,
                    keepdims=True)
        o_ref[...] = o - (m + jnp.log(s))

    fin = pl.pallas_call(
        fin_body, out_shape=jax.ShapeDtypeStruct((N, Hp), jnp.float32))

    return deg_kernel, edge_kernel, mm1, scale, mid, fin, NP, S, EP


def kernel(x, edge_index, W1, b1, W2, b2):
    N, D = x.shape
    E = edge_index.shape[1]
    H = W1.shape[1]
    C = W2.shape[1]
    Hp = 16
    (deg_kernel, edge_kernel, mm1, scale, mid, fin,
     NP, S, EP) = _build(N, E, D, H, C)

    src = edge_index[0].astype(jnp.int32)
    dst = edge_index[1].astype(jnp.int32)
    pad = EP - E
    lanes = jnp.arange(pad, dtype=jnp.int32) % 16
    srcp = jnp.concatenate([src, lanes]).reshape(NW, S, B)
    dstp = jnp.concatenate([dst, N + lanes]).reshape(NW, S, B)

    zn = jnp.zeros((NP,), jnp.float32)
    zr = jnp.zeros((NP, Hp), jnp.float32)
    W2p = jnp.concatenate(
        [W2, jnp.zeros((H, Hp - C), jnp.float32)], axis=1)
    b1r = b1.reshape(1, H)
    b2p = jnp.concatenate([b2, jnp.zeros((Hp - C,), jnp.float32)]).reshape(1, Hp)

    dpart = deg_kernel(dstp, zn)
    xw = mm1(x, W1)
    y1, dinv = scale(xw, dpart[0, :N, None], dpart[1, :N, None])
    a = edge_kernel(srcp, dstp, y1, zr)
    y2 = mid(a[0, :N], a[1, :N], y1, dinv, b1r, W2p)
    a2 = edge_kernel(srcp, dstp, y2, zr)
    out = fin(a2[0, :N], a2[1, :N], y2, dinv, b2p)
    return out[:, :C]
